# baseline (speedup 1.0000x reference)
import functools

import jax
import jax.numpy as jnp
from jax import lax
from jax.experimental import pallas as pl
from jax.experimental.pallas import tpu as pltpu
from jax.experimental.pallas import tpu_sc as plsc

TOTAL = 81920
DIM = 256
LANES = 16
VECS_PER_ROW = DIM // LANES

NUM_CORES = 2
NUM_SUBCORES = 16
NW = NUM_CORES * NUM_SUBCORES
ROWS_PER_W = TOTAL // NW
CHUNK = 128
NCHUNK = ROWS_PER_W // CHUNK
UNROLL = 4


def _rsqrt16(s):
    s = jnp.maximum(s, 1e-24)
    i = lax.bitcast_convert_type(s, jnp.int32)
    i = 0x5F3759DF - lax.shift_right_arithmetic(i, 1)
    y = lax.bitcast_convert_type(i, jnp.float32)
    h = 0.5 * s
    for _ in range(2):
        y = y * (1.5 - h * y * y)
    return y


def _allreduce16(v):
    lanes = lax.iota(jnp.int32, LANES)
    for k in (1, 2, 4, 8):
        idx = jnp.bitwise_xor(lanes, k)
        v = v + v.at[idx].get(mode="promise_in_bounds")
    return v


def _compute_chunk(buf):
    def row_body(g, carry2):
        for u in range(UNROLL):
            i = g * UNROLL + u
            vs = [buf[i, pl.ds(j * LANES, LANES)] for j in range(VECS_PER_ROW)]
            sq = [v * v for v in vs]
            while len(sq) > 1:
                sq = [sq[k] + sq[k + 1] for k in range(0, len(sq) - 1, 2)] + (
                    [sq[-1]] if len(sq) % 2 else []
                )
            r = _rsqrt16(_allreduce16(sq[0]))
            for j in range(VECS_PER_ROW):
                buf[i, pl.ds(j * LANES, LANES)] = vs[j] * r
        return carry2

    lax.fori_loop(0, CHUNK // UNROLL, row_body, 0)


def _sc_body(x_hbm, o_hbm, buf0, buf1, lsem0, lsem1, ssem0, ssem1):
    wid = lax.axis_index("s") * NUM_CORES + lax.axis_index("c")
    bufs = [buf0, buf1]
    lsems = [lsem0, lsem1]
    ssems = [ssem0, ssem1]

    def row0(c):
        return wid * ROWS_PER_W + c * CHUNK

    def load(c):
        b = c % 2
        return pltpu.async_copy(x_hbm.at[pl.ds(row0(c), CHUNK)], bufs[b], lsems[b])

    def store(c):
        b = c % 2
        return pltpu.async_copy(bufs[b], o_hbm.at[pl.ds(row0(c), CHUNK)], ssems[b])

    loads = {0: load(0)}
    stores = {}
    for c in range(NCHUNK):
        b = c % 2
        loads.pop(c).wait()
        if c + 1 < NCHUNK:
            if c - 1 in stores:
                stores.pop(c - 1).wait()
            loads[c + 1] = load(c + 1)
        _compute_chunk(bufs[b])
        stores[c] = store(c)
    for c in sorted(stores):
        stores.pop(c).wait()


def kernel(prototypes):
    mesh = plsc.VectorSubcoreMesh(core_axis_name="c", subcore_axis_name="s")
    return pl.kernel(
        _sc_body,
        mesh=mesh,
        out_type=jax.ShapeDtypeStruct((TOTAL, DIM), jnp.float32),
        scratch_types=[
            pltpu.VMEM((CHUNK, DIM), jnp.float32),
            pltpu.VMEM((CHUNK, DIM), jnp.float32),
            pltpu.SemaphoreType.DMA,
            pltpu.SemaphoreType.DMA,
            pltpu.SemaphoreType.DMA,
            pltpu.SemaphoreType.DMA,
        ],
        compiler_params=pltpu.CompilerParams(use_tc_tiling_on_sc=True),
    )(prototypes)


# speedup vs baseline: 1.4291x; 1.4291x over previous
import functools

import jax
import jax.numpy as jnp
from jax import lax
from jax.experimental import pallas as pl
from jax.experimental.pallas import tpu as pltpu
from jax.experimental.pallas import tpu_sc as plsc

TOTAL = 81920
DIM = 256
LANES = 16
VECS_PER_ROW = DIM // LANES

NUM_CORES = 2
NUM_SUBCORES = 16
NW = NUM_CORES * NUM_SUBCORES
ROWS_PER_W = TOTAL // NW
CHUNK = 128
NCHUNK = ROWS_PER_W // CHUNK
UNROLL = 2


def _rsqrt16(s):
    s = jnp.maximum(s, 1e-24)
    i = lax.bitcast_convert_type(s, jnp.int32)
    i = 0x5F3759DF - lax.shift_right_arithmetic(i, 1)
    y = lax.bitcast_convert_type(i, jnp.float32)
    h = 0.5 * s
    for _ in range(2):
        y = y * (1.5 - h * y * y)
    return y


def _allreduce16(v):
    lanes = lax.iota(jnp.int32, LANES)
    for k in (1, 2, 4, 8):
        idx = jnp.bitwise_xor(lanes, k)
        v = v + v.at[idx].get(mode="promise_in_bounds")
    return v


def _compute_chunk(buf):
    def row_body(g, carry2):
        for u in range(UNROLL):
            i = g * UNROLL + u
            vs = [buf[i, pl.ds(j * LANES, LANES)] for j in range(VECS_PER_ROW)]
            sq = [v * v for v in vs]
            while len(sq) > 1:
                sq = [sq[k] + sq[k + 1] for k in range(0, len(sq) - 1, 2)] + (
                    [sq[-1]] if len(sq) % 2 else []
                )
            r = _rsqrt16(_allreduce16(sq[0]))
            for j in range(VECS_PER_ROW):
                buf[i, pl.ds(j * LANES, LANES)] = vs[j] * r
        return carry2

    lax.fori_loop(0, CHUNK // UNROLL, row_body, 0)


def _sc_body(x_hbm, o_hbm, buf0, buf1, lsem0, lsem1, ssem0, ssem1):
    wid = lax.axis_index("s") * NUM_CORES + lax.axis_index("c")
    bufs = [buf0, buf1]
    lsems = [lsem0, lsem1]
    ssems = [ssem0, ssem1]

    def row0(c):
        return wid * ROWS_PER_W + c * CHUNK

    def load(c):
        b = c % 2
        return pltpu.async_copy(x_hbm.at[pl.ds(row0(c), CHUNK)], bufs[b], lsems[b])

    def store(c):
        b = c % 2
        return pltpu.async_copy(bufs[b], o_hbm.at[pl.ds(row0(c), CHUNK)], ssems[b])

    loads = {0: load(0)}
    stores = {}
    for c in range(NCHUNK):
        b = c % 2
        loads.pop(c).wait()
        if c + 1 < NCHUNK:
            if c - 1 in stores:
                stores.pop(c - 1).wait()
            loads[c + 1] = load(c + 1)
        _compute_chunk(bufs[b])
        stores[c] = store(c)
    for c in sorted(stores):
        stores.pop(c).wait()


def kernel(prototypes):
    mesh = plsc.VectorSubcoreMesh(core_axis_name="c", subcore_axis_name="s")
    return pl.kernel(
        _sc_body,
        mesh=mesh,
        out_type=jax.ShapeDtypeStruct((TOTAL, DIM), jnp.float32),
        scratch_types=[
            pltpu.VMEM((CHUNK, DIM), jnp.float32),
            pltpu.VMEM((CHUNK, DIM), jnp.float32),
            pltpu.SemaphoreType.DMA,
            pltpu.SemaphoreType.DMA,
            pltpu.SemaphoreType.DMA,
            pltpu.SemaphoreType.DMA,
        ],
        compiler_params=pltpu.CompilerParams(use_tc_tiling_on_sc=True),
    )(prototypes)


# speedup vs baseline: 1.6264x; 1.1381x over previous
import functools

import jax
import jax.numpy as jnp
from jax import lax
from jax.experimental import pallas as pl
from jax.experimental.pallas import tpu as pltpu
from jax.experimental.pallas import tpu_sc as plsc

TOTAL = 81920
DIM = 256
LANES = 16
VECS_PER_ROW = DIM // LANES

NUM_CORES = 2
NUM_SUBCORES = 16
NW = NUM_CORES * NUM_SUBCORES
ROWS_PER_W = TOTAL // NW
CHUNK = 128
NCHUNK = ROWS_PER_W // CHUNK
UNROLL = 2


def _rsqrt16(s):
    s = jnp.maximum(s, 1e-24)
    i = lax.bitcast_convert_type(s, jnp.int32)
    i = 0x5F3759DF - lax.shift_right_arithmetic(i, 1)
    y = lax.bitcast_convert_type(i, jnp.float32)
    h = 0.5 * s
    for _ in range(2):
        y = y * (1.5 - h * y * y)
    return y


def _allreduce16(v):
    lanes = lax.iota(jnp.int32, LANES)
    for k in (1, 2, 4, 8):
        idx = jnp.bitwise_xor(lanes, k)
        v = v + v.at[idx].get(mode="promise_in_bounds")
    return v


def _compute_chunk(buf):
    @plsc.parallel_loop(0, CHUNK, step=1, unroll=UNROLL)
    def _row(i):
        vs = [buf[i, pl.ds(j * LANES, LANES)] for j in range(VECS_PER_ROW)]
        sq = [v * v for v in vs]
        while len(sq) > 1:
            sq = [sq[k] + sq[k + 1] for k in range(0, len(sq) - 1, 2)] + (
                [sq[-1]] if len(sq) % 2 else []
            )
        r = _rsqrt16(_allreduce16(sq[0]))
        for j in range(VECS_PER_ROW):
            buf[i, pl.ds(j * LANES, LANES)] = vs[j] * r


def _sc_body(x_hbm, o_hbm, buf0, buf1, lsem0, lsem1, ssem0, ssem1):
    wid = lax.axis_index("s") * NUM_CORES + lax.axis_index("c")
    bufs = [buf0, buf1]
    lsems = [lsem0, lsem1]
    ssems = [ssem0, ssem1]

    def row0(c):
        return wid * ROWS_PER_W + c * CHUNK

    def load(c):
        b = c % 2
        return pltpu.async_copy(x_hbm.at[pl.ds(row0(c), CHUNK)], bufs[b], lsems[b])

    def store(c):
        b = c % 2
        return pltpu.async_copy(bufs[b], o_hbm.at[pl.ds(row0(c), CHUNK)], ssems[b])

    loads = {0: load(0)}
    stores = {}
    for c in range(NCHUNK):
        b = c % 2
        loads.pop(c).wait()
        if c + 1 < NCHUNK:
            if c - 1 in stores:
                stores.pop(c - 1).wait()
            loads[c + 1] = load(c + 1)
        _compute_chunk(bufs[b])
        stores[c] = store(c)
    for c in sorted(stores):
        stores.pop(c).wait()


def kernel(prototypes):
    mesh = plsc.VectorSubcoreMesh(core_axis_name="c", subcore_axis_name="s")
    return pl.kernel(
        _sc_body,
        mesh=mesh,
        out_type=jax.ShapeDtypeStruct((TOTAL, DIM), jnp.float32),
        scratch_types=[
            pltpu.VMEM((CHUNK, DIM), jnp.float32),
            pltpu.VMEM((CHUNK, DIM), jnp.float32),
            pltpu.SemaphoreType.DMA,
            pltpu.SemaphoreType.DMA,
            pltpu.SemaphoreType.DMA,
            pltpu.SemaphoreType.DMA,
        ],
        compiler_params=pltpu.CompilerParams(use_tc_tiling_on_sc=True),
    )(prototypes)


# speedup vs baseline: 1.6850x; 1.0360x over previous
import functools

import jax
import jax.numpy as jnp
from jax import lax
from jax.experimental import pallas as pl
from jax.experimental.pallas import tpu as pltpu
from jax.experimental.pallas import tpu_sc as plsc

TOTAL = 81920
DIM = 256
LANES = 16
VECS_PER_ROW = DIM // LANES

NUM_CORES = 2
NUM_SUBCORES = 16
NW = NUM_CORES * NUM_SUBCORES
ROWS_PER_W = TOTAL // NW
CHUNK = 128
NCHUNK = ROWS_PER_W // CHUNK
UNROLL = 2


def _rsqrt16(s):
    s = jnp.maximum(s, 1e-24)
    i = lax.bitcast_convert_type(s, jnp.int32)
    i = 0x5F3759DF - lax.shift_right_arithmetic(i, 1)
    y = lax.bitcast_convert_type(i, jnp.float32)
    h = 0.5 * s
    for _ in range(1):
        y = y * (1.5 - h * y * y)
    return y


def _allreduce16(v):
    lanes = lax.iota(jnp.int32, LANES)
    for k in (1, 2, 4, 8):
        idx = jnp.bitwise_xor(lanes, k)
        v = v + v.at[idx].get(mode="promise_in_bounds")
    return v


def _compute_chunk(buf):
    @plsc.parallel_loop(0, CHUNK, step=1, unroll=UNROLL)
    def _row(i):
        vs = [buf[i, pl.ds(j * LANES, LANES)] for j in range(VECS_PER_ROW)]
        sq = [v * v for v in vs]
        while len(sq) > 1:
            sq = [sq[k] + sq[k + 1] for k in range(0, len(sq) - 1, 2)] + (
                [sq[-1]] if len(sq) % 2 else []
            )
        r = _rsqrt16(_allreduce16(sq[0]))
        for j in range(VECS_PER_ROW):
            buf[i, pl.ds(j * LANES, LANES)] = vs[j] * r


def _sc_body(x_hbm, o_hbm, buf0, buf1, lsem0, lsem1, ssem0, ssem1):
    wid = lax.axis_index("s") * NUM_CORES + lax.axis_index("c")
    bufs = [buf0, buf1]
    lsems = [lsem0, lsem1]
    ssems = [ssem0, ssem1]

    def row0(c):
        return wid * ROWS_PER_W + c * CHUNK

    def load(c):
        b = c % 2
        return pltpu.async_copy(x_hbm.at[pl.ds(row0(c), CHUNK)], bufs[b], lsems[b])

    def store(c):
        b = c % 2
        return pltpu.async_copy(bufs[b], o_hbm.at[pl.ds(row0(c), CHUNK)], ssems[b])

    loads = {0: load(0)}
    stores = {}
    for c in range(NCHUNK):
        b = c % 2
        loads.pop(c).wait()
        if c + 1 < NCHUNK:
            if c - 1 in stores:
                stores.pop(c - 1).wait()
            loads[c + 1] = load(c + 1)
        _compute_chunk(bufs[b])
        stores[c] = store(c)
    for c in sorted(stores):
        stores.pop(c).wait()


def kernel(prototypes):
    mesh = plsc.VectorSubcoreMesh(core_axis_name="c", subcore_axis_name="s")
    return pl.kernel(
        _sc_body,
        mesh=mesh,
        out_type=jax.ShapeDtypeStruct((TOTAL, DIM), jnp.float32),
        scratch_types=[
            pltpu.VMEM((CHUNK, DIM), jnp.float32),
            pltpu.VMEM((CHUNK, DIM), jnp.float32),
            pltpu.SemaphoreType.DMA,
            pltpu.SemaphoreType.DMA,
            pltpu.SemaphoreType.DMA,
            pltpu.SemaphoreType.DMA,
        ],
        compiler_params=pltpu.CompilerParams(use_tc_tiling_on_sc=True),
    )(prototypes)


# speedup vs baseline: 1.8565x; 1.1018x over previous
import functools

import jax
import jax.numpy as jnp
from jax import lax
from jax.experimental import pallas as pl
from jax.experimental.pallas import tpu as pltpu
from jax.experimental.pallas import tpu_sc as plsc

TOTAL = 81920
DIM = 256
LANES = 16
VECS_PER_ROW = DIM // LANES

TC_BLOCK = 4096
TC_ROWS = 53248
SC_ROWS = TOTAL - TC_ROWS

NUM_CORES = 2
NUM_SUBCORES = 16
NW = NUM_CORES * NUM_SUBCORES
ROWS_PER_W = SC_ROWS // NW
CHUNK = 128
NCHUNK = ROWS_PER_W // CHUNK
UNROLL = 2


def _rsqrt16(s):
    s = jnp.maximum(s, 1e-24)
    i = lax.bitcast_convert_type(s, jnp.int32)
    i = 0x5F3759DF - lax.shift_right_arithmetic(i, 1)
    y = lax.bitcast_convert_type(i, jnp.float32)
    h = 0.5 * s
    for _ in range(1):
        y = y * (1.5 - h * y * y)
    return y


def _allreduce16(v):
    lanes = lax.iota(jnp.int32, LANES)
    for k in (1, 2, 4, 8):
        idx = jnp.bitwise_xor(lanes, k)
        v = v + v.at[idx].get(mode="promise_in_bounds")
    return v


def _compute_chunk(buf):
    @plsc.parallel_loop(0, CHUNK, step=1, unroll=UNROLL)
    def _row(i):
        vs = [buf[i, pl.ds(j * LANES, LANES)] for j in range(VECS_PER_ROW)]
        sq = [v * v for v in vs]
        while len(sq) > 1:
            sq = [sq[k] + sq[k + 1] for k in range(0, len(sq) - 1, 2)] + (
                [sq[-1]] if len(sq) % 2 else []
            )
        r = _rsqrt16(_allreduce16(sq[0]))
        for j in range(VECS_PER_ROW):
            buf[i, pl.ds(j * LANES, LANES)] = vs[j] * r


def _sc_body(x_hbm, o_hbm, buf0, buf1, lsem0, lsem1, ssem0, ssem1):
    wid = lax.axis_index("s") * NUM_CORES + lax.axis_index("c")
    bufs = [buf0, buf1]
    lsems = [lsem0, lsem1]
    ssems = [ssem0, ssem1]

    def in_row0(c):
        return TC_ROWS + wid * ROWS_PER_W + c * CHUNK

    def out_row0(c):
        return wid * ROWS_PER_W + c * CHUNK

    def load(c):
        b = c % 2
        return pltpu.async_copy(x_hbm.at[pl.ds(in_row0(c), CHUNK)], bufs[b], lsems[b])

    def store(c):
        b = c % 2
        return pltpu.async_copy(bufs[b], o_hbm.at[pl.ds(out_row0(c), CHUNK)], ssems[b])

    loads = {0: load(0)}
    stores = {}
    for c in range(NCHUNK):
        b = c % 2
        loads.pop(c).wait()
        if c + 1 < NCHUNK:
            if c - 1 in stores:
                stores.pop(c - 1).wait()
            loads[c + 1] = load(c + 1)
        _compute_chunk(bufs[b])
        stores[c] = store(c)
    for c in sorted(stores):
        stores.pop(c).wait()


def _sc_normalize_tail(x):
    mesh = plsc.VectorSubcoreMesh(core_axis_name="c", subcore_axis_name="s")
    return pl.kernel(
        _sc_body,
        mesh=mesh,
        out_type=jax.ShapeDtypeStruct((SC_ROWS, DIM), jnp.float32),
        scratch_types=[
            pltpu.VMEM((CHUNK, DIM), jnp.float32),
            pltpu.VMEM((CHUNK, DIM), jnp.float32),
            pltpu.SemaphoreType.DMA,
            pltpu.SemaphoreType.DMA,
            pltpu.SemaphoreType.DMA,
            pltpu.SemaphoreType.DMA,
        ],
        compiler_params=pltpu.CompilerParams(use_tc_tiling_on_sc=True),
    )(x)


def _tc_norm_body(x_ref, o_ref):
    x = x_ref[...]
    s = jnp.sum(x * x, axis=-1, keepdims=True)
    o_ref[...] = x * jax.lax.rsqrt(jnp.maximum(s, 1e-24))


def _tc_normalize_head(x):
    return pl.pallas_call(
        _tc_norm_body,
        grid=(TC_ROWS // TC_BLOCK,),
        in_specs=[pl.BlockSpec((TC_BLOCK, DIM), lambda i: (i, 0))],
        out_specs=pl.BlockSpec((TC_BLOCK, DIM), lambda i: (i, 0)),
        out_shape=jax.ShapeDtypeStruct((TOTAL, DIM), jnp.float32),
    )(x)


def _tc_copy_body(s_ref, _, o_ref):
    o_ref[...] = s_ref[...]


def _merge_tail(sc_part, head_out):
    nblk = SC_ROWS // TC_BLOCK
    off = TC_ROWS // TC_BLOCK
    return pl.pallas_call(
        _tc_copy_body,
        grid=(nblk,),
        in_specs=[
            pl.BlockSpec((TC_BLOCK, DIM), lambda i: (i, 0)),
            pl.BlockSpec((TC_BLOCK, DIM), lambda i: (i + off, 0)),
        ],
        out_specs=pl.BlockSpec((TC_BLOCK, DIM), lambda i: (i + off, 0)),
        out_shape=jax.ShapeDtypeStruct((TOTAL, DIM), jnp.float32),
        input_output_aliases={1: 0},
    )(sc_part, head_out)


def kernel(prototypes):
    sc_part = _sc_normalize_tail(prototypes)
    head = _tc_normalize_head(prototypes)
    return _merge_tail(sc_part, head)
